# baseline (speedup 1.0000x reference)
import functools

import jax
import jax.numpy as jnp
from jax import lax
from jax.experimental import pallas as pl
from jax.experimental.pallas import tpu as pltpu
from jax.experimental.pallas import tpu_sc as plsc

VOCAB = 100000
EMBED_DIM = 1024
BATCH = 4
SEQ = 4096

NC = 2
NS = 16
NW = NC * NS
S_PER_W = SEQ // NW
CS = 8
NCHUNK = S_PER_W // CS
ROWS = BATCH * CS
NBUF = 3
LANES = 16
VPR = EMBED_DIM // LANES


def _body(idx_hbm, wte_hbm, wpe_hbm, out_hbm, idx_v, rows_v, pos_v, *sems):
    gsem = sems[0:NBUF]
    psem = sems[NBUF:2 * NBUF]
    ssem = sems[2 * NBUF:3 * NBUF]

    wid = lax.axis_index("s") * NC + lax.axis_index("c")
    s0 = wid * S_PER_W

    pltpu.sync_copy(idx_hbm.at[wid], idx_v)

    def clamp_row(r, carry):
        for j in range(ROWS // LANES):
            sl = pl.ds(j * LANES, LANES)
            idx_v[r, sl] = jnp.minimum(idx_v[r, sl], VOCAB - 1)
        return carry
    lax.fori_loop(0, NCHUNK, clamp_row, 0)

    def issue(c, p):
        g = pltpu.async_copy(wte_hbm.at[idx_v.at[c]], rows_v.at[p], gsem[p])
        q = pltpu.async_copy(wpe_hbm.at[pl.ds(s0 + c * CS, CS)],
                             pos_v.at[p], psem[p])
        return g, q

    pending = {}
    stores = {}
    for c in range(min(2, NCHUNK)):
        pending[c] = issue(c, c % NBUF)

    for c in range(NCHUNK):
        p = c % NBUF
        g, q = pending.pop(c)
        g.wait()
        q.wait()

        nxt = c + 2
        if nxt < NCHUNK:
            np_ = nxt % NBUF
            for h in stores.pop(np_, []):
                h.wait()
            pending[nxt] = issue(nxt, np_)

        def add_pos(r, carry, _p=p):
            def jblk(jb, carry2):
                for jj in range(4):
                    sl = pl.ds(jb * 4 * LANES + jj * LANES, LANES)
                    pv = pos_v[_p, r, sl]
                    for b in range(BATCH):
                        plsc.addupdate(rows_v.at[_p, b * CS + r, sl], pv)
                return carry2
            lax.fori_loop(0, VPR // 4, jblk, 0)
            return carry
        lax.fori_loop(0, CS, add_pos, 0)

        stores[p] = [
            pltpu.async_copy(
                rows_v.at[p, pl.ds(b * CS, CS)],
                out_hbm.at[pl.ds(b * SEQ + s0 + c * CS, CS)],
                ssem[p])
            for b in range(BATCH)
        ]

    for hs in stores.values():
        for h in hs:
            h.wait()


@jax.jit
def kernel(inputs, wte, wpe):
    idx = (inputs.astype(jnp.int32)
           .reshape(BATCH, NW, NCHUNK, CS)
           .transpose(1, 2, 0, 3)
           .reshape(NW, NCHUNK, ROWS))
    run = functools.partial(
        pl.kernel,
        mesh=plsc.VectorSubcoreMesh(core_axis_name="c", subcore_axis_name="s"),
        out_type=jax.ShapeDtypeStruct((BATCH * SEQ, EMBED_DIM), jnp.float32),
        scratch_types=[
            pltpu.VMEM((NCHUNK, ROWS), jnp.int32),
            pltpu.VMEM((NBUF, ROWS, EMBED_DIM), jnp.float32),
            pltpu.VMEM((NBUF, CS, EMBED_DIM), jnp.float32),
        ] + [pltpu.SemaphoreType.DMA] * (3 * NBUF),
    )(_body)
    out = run(idx, wte, wpe)
    return out.reshape(BATCH, SEQ, EMBED_DIM)


# speedup vs baseline: 1.0645x; 1.0645x over previous
import functools

import jax
import jax.numpy as jnp
from jax import lax
from jax.experimental import pallas as pl
from jax.experimental.pallas import tpu as pltpu
from jax.experimental.pallas import tpu_sc as plsc

VOCAB = 100000
EMBED_DIM = 1024
BATCH = 4
SEQ = 4096

NC = 2
NS = 16
NW = NC * NS
S_PER_W = SEQ // NW
CS = 8
NCHUNK = S_PER_W // CS
ROWS = BATCH * CS
NBUF = 3
LANES = 16
VPR = EMBED_DIM // LANES


def _body(idx_hbm, wte_hbm, wpe_hbm, out_hbm, idx_v, rows_v, pos_v, *sems):
    gsem = sems[0:NBUF]
    psem = sems[NBUF:2 * NBUF]
    ssem = sems[2 * NBUF:3 * NBUF]

    wid = lax.axis_index("s") * NC + lax.axis_index("c")
    s0 = wid * S_PER_W

    pltpu.sync_copy(idx_hbm.at[wid], idx_v)

    def clamp_row(r, carry):
        for j in range(ROWS // LANES):
            sl = pl.ds(j * LANES, LANES)
            idx_v[r, sl] = jnp.minimum(idx_v[r, sl], VOCAB - 1)
        return carry
    lax.fori_loop(0, NCHUNK, clamp_row, 0)

    def issue(c, p):
        g = pltpu.async_copy(wte_hbm.at[idx_v.at[c]], rows_v.at[p], gsem[p])
        q = pltpu.async_copy(wpe_hbm.at[pl.ds(s0 + c * CS, CS)],
                             pos_v.at[p], psem[p])
        return g, q

    pending = {}
    stores = {}
    for c in range(min(2, NCHUNK)):
        pending[c] = issue(c, c % NBUF)

    for c in range(NCHUNK):
        p = c % NBUF
        g, q = pending.pop(c)
        g.wait()
        q.wait()

        nxt = c + 2
        if nxt < NCHUNK:
            np_ = nxt % NBUF
            for h in stores.pop(np_, []):
                h.wait()
            pending[nxt] = issue(nxt, np_)

        def add_pos(r, carry, _p=p):
            for j in range(VPR):
                sl = pl.ds(j * LANES, LANES)
                pv = pos_v[_p, r, sl]
                for b in range(BATCH):
                    rows_v[_p, b * CS + r, sl] = (
                        rows_v[_p, b * CS + r, sl] + pv)
            return carry
        lax.fori_loop(0, CS, add_pos, 0)

        stores[p] = [
            pltpu.async_copy(
                rows_v.at[p, pl.ds(b * CS, CS)],
                out_hbm.at[pl.ds(b * SEQ + s0 + c * CS, CS)],
                ssem[p])
            for b in range(BATCH)
        ]

    for hs in stores.values():
        for h in hs:
            h.wait()


@jax.jit
def kernel(inputs, wte, wpe):
    idx = (inputs.astype(jnp.int32)
           .reshape(BATCH, NW, NCHUNK, CS)
           .transpose(1, 2, 0, 3)
           .reshape(NW, NCHUNK, ROWS))
    run = functools.partial(
        pl.kernel,
        mesh=plsc.VectorSubcoreMesh(core_axis_name="c", subcore_axis_name="s"),
        out_type=jax.ShapeDtypeStruct((BATCH * SEQ, EMBED_DIM), jnp.float32),
        scratch_types=[
            pltpu.VMEM((NCHUNK, ROWS), jnp.int32),
            pltpu.VMEM((NBUF, ROWS, EMBED_DIM), jnp.float32),
            pltpu.VMEM((NBUF, CS, EMBED_DIM), jnp.float32),
        ] + [pltpu.SemaphoreType.DMA] * (3 * NBUF),
    )(_body)
    out = run(idx, wte, wpe)
    return out.reshape(BATCH, SEQ, EMBED_DIM)
